# speedup vs baseline: 1.7584x; 1.7584x over previous
import jax
import jax.numpy as jnp
import numpy as np
from jax.experimental import pallas as pl
from jax.experimental.pallas import tpu as pltpu

BASE_GSD_ = 10.0
EMBED_ = 128
D4_ = 32
MAX_SEQ_ = 24
B_, H_, W_, T_, CG_ = 16, 16, 16, 12, 4
TCG_ = T_ * CG_


def _pos_table():
    omega = jnp.arange(D4_ // 2, dtype=jnp.float32) / (D4_ / 2.0)
    omega = 1.0 / (10000.0 ** omega)
    out = jnp.arange(MAX_SEQ_, dtype=jnp.float32)[:, None] * omega[None, :]
    return jnp.concatenate([jnp.sin(out), jnp.cos(out)], axis=1)[:T_]


def _month_table():
    angles = jnp.arange(0, 13, dtype=jnp.float32) / (12.0 / (2.0 * np.pi))
    sin_t = jnp.stack([jnp.sin(angles)] * (D4_ // 2), axis=-1)
    cos_t = jnp.stack([jnp.cos(angles)] * (D4_ // 2), axis=-1)
    return jnp.concatenate([sin_t[:-1], cos_t[:-1]], axis=-1)


def _spatial_table(gsd_ratio):
    grid_h = jnp.arange(H_, dtype=jnp.float32)
    grid_w = jnp.arange(W_, dtype=jnp.float32)
    gw, gh = jnp.meshgrid(grid_w, grid_h, indexing="xy")
    d = D4_ // 2
    omega = jnp.arange(d // 2, dtype=jnp.float32) / (d / 2.0)
    omega = 1.0 / (10000.0 ** omega)

    def sincos(pos):
        out = (pos * gsd_ratio).reshape(-1)[:, None] * omega[None, :]
        return jnp.concatenate([jnp.sin(out), jnp.cos(out)], axis=1)

    emb = jnp.concatenate([sincos(gw), sincos(gh)], axis=1)
    return emb.reshape(H_, W_, D4_)


def _encoder_kernel(months_ref, s2_ref, base_ref, mtab_ref, spat_ref, out_ref):
    m = months_ref[0]
    lane = jax.lax.broadcasted_iota(jnp.int32, (T_, 128), 1)
    onehot = (m == lane).astype(jnp.float32)
    mb = jnp.dot(onehot, mtab_ref[...], preferred_element_type=jnp.float32)
    mb48 = jnp.broadcast_to(mb[:, None, :], (T_, CG_, 128)).reshape(TCG_, 128)
    bias = base_ref[...] + mb48
    out_ref[0, 0] = s2_ref[0, 0] + bias[None, :, :] + spat_ref[0][:, None, :]


def kernel(s2, timestamps, channel_embeds, patch_size, input_res):
    b, h, w, t, cg, e = s2.shape
    s2r = s2.reshape(b, h, w, t * cg, e)

    months = timestamps[:, 1, :].astype(jnp.int32)
    months_vm = jnp.broadcast_to(months[:, :, None], (b, t, 128)).astype(jnp.int32)

    ch = jnp.broadcast_to(channel_embeds[None, :, :], (t, cg, D4_))
    pos = jnp.broadcast_to(_pos_table()[:, None, :], (t, cg, D4_))
    zeros = jnp.zeros((t, cg, 2 * D4_), jnp.float32)
    base = jnp.concatenate([ch, pos, zeros], axis=-1).reshape(t * cg, 128)

    mt = _month_table()
    mtab = jnp.zeros((128, 128), jnp.float32)
    mtab = jax.lax.dynamic_update_slice(mtab, mt, (0, 2 * D4_))

    gsd_ratio = (input_res * patch_size) / BASE_GSD_
    spat = _spatial_table(gsd_ratio)
    spat128 = jnp.concatenate(
        [jnp.zeros((h, w, 3 * D4_), jnp.float32), spat], axis=-1)

    out = pl.pallas_call(
        _encoder_kernel,
        grid=(b, h),
        in_specs=[
            pl.BlockSpec((1, t, 128), lambda i, j: (i, 0, 0)),
            pl.BlockSpec((1, 1, w, t * cg, 128), lambda i, j: (i, j, 0, 0, 0)),
            pl.BlockSpec((t * cg, 128), lambda i, j: (0, 0)),
            pl.BlockSpec((128, 128), lambda i, j: (0, 0)),
            pl.BlockSpec((1, w, 128), lambda i, j: (j, 0, 0)),
        ],
        out_specs=pl.BlockSpec((1, 1, w, t * cg, 128), lambda i, j: (i, j, 0, 0, 0)),
        out_shape=jax.ShapeDtypeStruct((b, h, w, t * cg, 128), jnp.float32),
        compiler_params=pltpu.CompilerParams(
            dimension_semantics=("parallel", "parallel"),
        ),
    )(months_vm, s2r, base, mtab, spat128)
    return out.reshape(b, h, w, t, cg, e)


# speedup vs baseline: 4.7992x; 2.7293x over previous
import jax
import jax.numpy as jnp
import numpy as np
from jax.experimental import pallas as pl
from jax.experimental.pallas import tpu as pltpu

BASE_GSD_ = 10.0
EMBED_ = 128
D4_ = 32
MAX_SEQ_ = 24
B_, H_, W_, T_, CG_ = 16, 16, 16, 12, 4
TCG_ = T_ * CG_


def _pos_table():
    omega = jnp.arange(D4_ // 2, dtype=jnp.float32) / (D4_ / 2.0)
    omega = 1.0 / (10000.0 ** omega)
    out = jnp.arange(MAX_SEQ_, dtype=jnp.float32)[:, None] * omega[None, :]
    return jnp.concatenate([jnp.sin(out), jnp.cos(out)], axis=1)[:T_]


def _month_table():
    angles = jnp.arange(0, 13, dtype=jnp.float32) / (12.0 / (2.0 * np.pi))
    sin_t = jnp.stack([jnp.sin(angles)] * (D4_ // 2), axis=-1)
    cos_t = jnp.stack([jnp.cos(angles)] * (D4_ // 2), axis=-1)
    return jnp.concatenate([sin_t[:-1], cos_t[:-1]], axis=-1)


def _spatial_table(gsd_ratio):
    grid_h = jnp.arange(H_, dtype=jnp.float32)
    grid_w = jnp.arange(W_, dtype=jnp.float32)
    gw, gh = jnp.meshgrid(grid_w, grid_h, indexing="xy")
    d = D4_ // 2
    omega = jnp.arange(d // 2, dtype=jnp.float32) / (d / 2.0)
    omega = 1.0 / (10000.0 ** omega)

    def sincos(pos):
        out = (pos * gsd_ratio).reshape(-1)[:, None] * omega[None, :]
        return jnp.concatenate([jnp.sin(out), jnp.cos(out)], axis=1)

    emb = jnp.concatenate([sincos(gw), sincos(gh)], axis=1)
    return emb.reshape(H_, W_, D4_)


def _encoder_kernel(months_ref, s2_ref, base_ref, mtab_ref, spat_ref, out_ref):
    m = months_ref[0]
    lane = jax.lax.broadcasted_iota(jnp.int32, (T_, 128), 1)
    onehot = (m == lane).astype(jnp.float32)
    mb = jnp.dot(onehot, mtab_ref[...], preferred_element_type=jnp.float32)
    mb48 = jnp.broadcast_to(mb[:, None, :], (T_, CG_, 128)).reshape(TCG_, 128)
    bias = base_ref[...] + mb48
    out_ref[0] = s2_ref[0] + bias[None, None, :, :] + spat_ref[...][:, :, None, :]


def kernel(s2, timestamps, channel_embeds, patch_size, input_res):
    b, h, w, t, cg, e = s2.shape
    s2r = s2.reshape(b, h, w, t * cg, e)

    months = timestamps[:, 1, :].astype(jnp.int32)
    months_vm = jnp.broadcast_to(months[:, :, None], (b, t, 128)).astype(jnp.int32)

    ch = jnp.broadcast_to(channel_embeds[None, :, :], (t, cg, D4_))
    pos = jnp.broadcast_to(_pos_table()[:, None, :], (t, cg, D4_))
    zeros = jnp.zeros((t, cg, 2 * D4_), jnp.float32)
    base = jnp.concatenate([ch, pos, zeros], axis=-1).reshape(t * cg, 128)

    mt = _month_table()
    mtab = jnp.zeros((128, 128), jnp.float32)
    mtab = jax.lax.dynamic_update_slice(mtab, mt, (0, 2 * D4_))

    gsd_ratio = (input_res * patch_size) / BASE_GSD_
    spat = _spatial_table(gsd_ratio)
    spat128 = jnp.concatenate(
        [jnp.zeros((h, w, 3 * D4_), jnp.float32), spat], axis=-1)

    out = pl.pallas_call(
        _encoder_kernel,
        grid=(b,),
        in_specs=[
            pl.BlockSpec((1, t, 128), lambda i: (i, 0, 0)),
            pl.BlockSpec((1, h, w, t * cg, 128), lambda i: (i, 0, 0, 0, 0)),
            pl.BlockSpec((t * cg, 128), lambda i: (0, 0)),
            pl.BlockSpec((128, 128), lambda i: (0, 0)),
            pl.BlockSpec((h, w, 128), lambda i: (0, 0, 0)),
        ],
        out_specs=pl.BlockSpec((1, h, w, t * cg, 128), lambda i: (i, 0, 0, 0, 0)),
        out_shape=jax.ShapeDtypeStruct((b, h, w, t * cg, 128), jnp.float32),
        compiler_params=pltpu.CompilerParams(
            dimension_semantics=("parallel",),
        ),
    )(months_vm, s2r, base, mtab, spat128)
    return out.reshape(b, h, w, t, cg, e)
